# baseline (speedup 1.0000x reference)
import functools

import jax
import jax.numpy as jnp
from jax import lax
from jax.experimental import pallas as pl
from jax.experimental.pallas import tpu as pltpu
from jax.experimental.pallas import tpu_sc as plsc

BATCH = 16384
EMBED_DIM = 16
LANES = 16
CHUNK = 128
ROWS_PER_WIDE = 8


def _make_sc_kernel(b_per_w: int):
    n_chunks = b_per_w // CHUNK
    groups = CHUNK // LANES
    mesh = plsc.VectorSubcoreMesh(core_axis_name="c", subcore_axis_name="s")

    @functools.partial(
        pl.kernel,
        mesh=mesh,
        out_type=jax.ShapeDtypeStruct((BATCH,), jnp.float32),
        compiler_params=pltpu.CompilerParams(needs_layout_passes=False),
        scratch_types=[
            pltpu.VMEM((n_chunks, CHUNK), jnp.int32),
            pltpu.VMEM((n_chunks, CHUNK), jnp.int32),
            pltpu.VMEM((n_chunks, CHUNK), jnp.int32),
            pltpu.VMEM((n_chunks, CHUNK), jnp.int32),
            pltpu.VMEM((CHUNK, 128), jnp.float32),
            pltpu.VMEM((CHUNK, 128), jnp.float32),
            pltpu.VMEM((b_per_w,), jnp.float32),
            pltpu.SemaphoreType.DMA,
        ],
    )
    def sc_kernel(user_hbm, songs_hbm, utab_hbm, stab_hbm, out_hbm,
                  uidx, sidx, uwidx, swidx, uwide, swide, outv, sem):
        num_cores = 2
        wid = lax.axis_index("s") * num_cores + lax.axis_index("c")
        base = wid * b_per_w

        for c in range(n_chunks):
            pltpu.sync_copy(user_hbm.at[pl.ds(base + c * CHUNK, CHUNK)],
                            uidx.at[c])
            pltpu.sync_copy(songs_hbm.at[pl.ds(base + c * CHUNK, CHUNK)],
                            sidx.at[c])
            for v in range(CHUNK // LANES):
                sl = pl.ds(v * LANES, LANES)
                uwidx[c, sl] = lax.shift_right_logical(uidx[c, sl], 3)
                swidx[c, sl] = lax.shift_right_logical(sidx[c, sl], 3)

        riota = lax.iota(jnp.int32, LANES)

        for c in range(n_chunks):
            cp_u = pltpu.async_copy(utab_hbm.at[uwidx.at[c]], uwide, sem)
            cp_s = pltpu.async_copy(stab_hbm.at[swidx.at[c]], swide, sem)
            cp_u.wait()
            cp_s.wait()

            for g in range(groups):
                sl = pl.ds(g * LANES, LANES)
                rvec = riota + g * LANES
                uoff = lax.shift_left(jnp.bitwise_and(uidx[c, sl], 7), 4)
                soff = lax.shift_left(jnp.bitwise_and(sidx[c, sl], 7), 4)
                acc = jnp.zeros((LANES,), jnp.float32)
                for d in range(EMBED_DIM):
                    uc = plsc.load_gather(uwide, [rvec, uoff + d])
                    sc = plsc.load_gather(swide, [rvec, soff + d])
                    acc = acc + uc * sc
                outv[pl.ds(c * CHUNK + g * LANES, LANES)] = acc

        pltpu.sync_copy(outv, out_hbm.at[pl.ds(base, b_per_w)])

    return sc_kernel


def kernel(user, songs, user_table, song_table):
    info = plsc.get_sparse_core_info()
    num_workers = info.num_cores * info.num_subcores
    b_per_w = BATCH // num_workers
    sc = _make_sc_kernel(b_per_w)
    n_users, dim = user_table.shape
    n_songs, _ = song_table.shape
    utab_w = user_table.reshape(n_users // ROWS_PER_WIDE, dim * ROWS_PER_WIDE)
    stab_w = song_table.reshape(n_songs // ROWS_PER_WIDE, dim * ROWS_PER_WIDE)
    out = sc(user.reshape(BATCH).astype(jnp.int32),
             songs.reshape(BATCH).astype(jnp.int32),
             utab_w, stab_w)
    return out.reshape(BATCH, 1)


# speedup vs baseline: 1.0173x; 1.0173x over previous
import functools

import jax
import jax.numpy as jnp
from jax import lax
from jax.experimental import pallas as pl
from jax.experimental.pallas import tpu as pltpu
from jax.experimental.pallas import tpu_sc as plsc

BATCH = 16384
EMBED_DIM = 16
LANES = 16
CHUNK = 128


def _make_sc_kernel(b_per_w: int):
    n_chunks = b_per_w // CHUNK
    mesh = plsc.VectorSubcoreMesh(core_axis_name="c", subcore_axis_name="s")

    @functools.partial(
        pl.kernel,
        mesh=mesh,
        out_type=jax.ShapeDtypeStruct((BATCH,), jnp.float32),
        compiler_params=pltpu.CompilerParams(
            needs_layout_passes=False, use_tc_tiling_on_sc=False),
        scratch_types=[
            pltpu.VMEM((n_chunks, CHUNK), jnp.int32),
            pltpu.VMEM((n_chunks, CHUNK), jnp.int32),
            pltpu.VMEM((b_per_w, EMBED_DIM), jnp.float32),
            pltpu.VMEM((b_per_w, EMBED_DIM), jnp.float32),
            pltpu.VMEM((b_per_w,), jnp.float32),
            pltpu.SemaphoreType.DMA,
        ],
    )
    def sc_kernel(user_hbm, songs_hbm, utab_hbm, stab_hbm, out_hbm,
                  uidx, sidx, urows, srows, outv, sem):
        num_cores = 2
        wid = lax.axis_index("s") * num_cores + lax.axis_index("c")
        base = wid * b_per_w

        for c in range(n_chunks):
            pltpu.sync_copy(user_hbm.at[pl.ds(base + c * CHUNK, CHUNK)],
                            uidx.at[c])
            pltpu.sync_copy(songs_hbm.at[pl.ds(base + c * CHUNK, CHUNK)],
                            sidx.at[c])

        copies = []
        for c in range(n_chunks):
            copies.append(pltpu.async_copy(
                utab_hbm.at[uidx.at[c]],
                urows.at[pl.ds(c * CHUNK, CHUNK)], sem))
            copies.append(pltpu.async_copy(
                stab_hbm.at[sidx.at[c]],
                srows.at[pl.ds(c * CHUNK, CHUNK)], sem))
        for cp in copies:
            cp.wait()

        riota = lax.iota(jnp.int32, LANES)

        def g_body(g, carry):
            ridx = riota + g * LANES
            acc = jnp.zeros((LANES,), jnp.float32)
            for d in range(EMBED_DIM):
                dv = jnp.full((LANES,), d, jnp.int32)
                uc = plsc.load_gather(urows, [ridx, dv])
                sc = plsc.load_gather(srows, [ridx, dv])
                acc = acc + uc * sc
            plsc.store_scatter(outv, [ridx], acc)
            return carry

        lax.fori_loop(0, b_per_w // LANES, g_body, 0)

        pltpu.sync_copy(outv, out_hbm.at[pl.ds(base, b_per_w)])

    return sc_kernel


def kernel(user, songs, user_table, song_table):
    info = plsc.get_sparse_core_info()
    num_workers = info.num_cores * info.num_subcores
    b_per_w = BATCH // num_workers
    sc = _make_sc_kernel(b_per_w)
    out = sc(user.reshape(BATCH).astype(jnp.int32),
             songs.reshape(BATCH).astype(jnp.int32),
             user_table, song_table)
    return out.reshape(BATCH, 1)
